# baseline (speedup 1.0000x reference)
import functools

import jax
import jax.numpy as jnp
from jax import lax
from jax.experimental import pallas as pl
from jax.experimental.pallas import tpu as pltpu
from jax.experimental.pallas import tpu_sc as plsc

N = 10000
E = 40000
IN_DIM = 512
EMB = 3000
NUM_GRAPHS = 16
K = 50
OUT = 33

NP = 10240
F = 128
C = 24
EMB_P = C * F
NC = 2
NT = 16
EPT = 2560
EP = NT * EPT
NB = EPT // 128
NBH = NB // 2
SLICE = NP // NT
TAB = 512
GP = 24

_mesh = plsc.VectorSubcoreMesh(core_axis_name="c", subcore_axis_name="s")
_SC_PARAMS = pltpu.CompilerParams(needs_layout_passes=False)


def _i16(v):
    return jnp.full((16,), v, dtype=jnp.int32)


def _k1_body(row_hbm, col_hbm, tab_hbm, norm_hbm, invdeg_hbm,
             acc_sh, rowv, colv, tabv, degv, disv, normb, onesv, zerosv, invb):
    tid = lax.axis_index("s")
    cid = lax.axis_index("c")
    pltpu.sync_copy(row_hbm.at[tid], rowv)
    pltpu.sync_copy(col_hbm.at[tid], colv)
    pltpu.sync_copy(tab_hbm, tabv)

    def fill(i, _):
        onesv[pl.ds(i * 16, 16)] = jnp.full((16,), 1.0, jnp.float32)
        return 0
    lax.fori_loop(0, 8, fill, 0)

    def fillz(i, _):
        zerosv[pl.ds(i * 16, 16)] = jnp.zeros((16,), jnp.float32)
        return 0
    lax.fori_loop(0, SLICE // 16, fillz, 0)

    pltpu.sync_copy(zerosv, acc_sh.at[pl.ds(tid * SLICE, SLICE)])
    plsc.subcore_barrier()

    def cnt(b, _):
        pltpu.sync_copy(onesv, acc_sh.at[colv.at[b]], add=True)
        return 0
    lax.fori_loop(0, NB, cnt, 0)
    plsc.subcore_barrier()

    pltpu.sync_copy(acc_sh, degv)

    def mkdis(i, _):
        d = degv[pl.ds(i * 16, 16)] + 1.0
        di = jnp.minimum(d.astype(jnp.int32), TAB - 1)
        disv[pl.ds(i * 16, 16)] = plsc.load_gather(tabv, [di])
        return 0
    lax.fori_loop(0, NP // 16, mkdis, 0)

    def mknorm(bi, _):
        b = cid * NBH + bi
        for g in range(8):
            sl = pl.ds(g * 16, 16)
            r16 = rowv[b, sl]
            c16 = colv[b, sl]
            normb[pl.ds(bi * 128 + g * 16, 16)] = (
                plsc.load_gather(disv, [r16]) *
                plsc.load_gather(disv, [c16]))
        return 0
    lax.fori_loop(0, NBH, mknorm, 0)
    pltpu.sync_copy(normb,
                    norm_hbm.at[pl.ds(tid * EPT + cid * (NBH * 128),
                                      NBH * 128)])

    @pl.when(cid == 0)
    def _():
        def mkinv(i, _):
            d = degv[pl.ds(tid * SLICE + i * 16, 16)] + 1.0
            invb[pl.ds(i * 16, 16)] = 1.0 / d
            return 0
        lax.fori_loop(0, SLICE // 16, mkinv, 0)
        pltpu.sync_copy(invb, invdeg_hbm.at[pl.ds(tid * SLICE, SLICE)])


_k1 = pl.kernel(
    _k1_body,
    out_type=(
        jax.ShapeDtypeStruct((EP,), jnp.float32),
        jax.ShapeDtypeStruct((NP,), jnp.float32),
    ),
    mesh=_mesh,
    compiler_params=_SC_PARAMS,
    scratch_types=[
        pltpu.VMEM_SHARED((NP,), jnp.float32),
        pltpu.VMEM((NB, 128), jnp.int32),
        pltpu.VMEM((NB, 128), jnp.int32),
        pltpu.VMEM((TAB,), jnp.float32),
        pltpu.VMEM((NP,), jnp.float32),
        pltpu.VMEM((NP,), jnp.float32),
        pltpu.VMEM((NBH * 128,), jnp.float32),
        pltpu.VMEM((128,), jnp.float32),
        pltpu.VMEM((SLICE,), jnp.float32),
        pltpu.VMEM((SLICE,), jnp.float32),
    ],
)


def _conv_body(layer, *refs):
    if layer == 0:
        (h_hbm, norm_hbm, row_hbm, col_hbm, invdeg_hbm, root_hbm,
         out_hbm, ssum_hbm, ssq_hbm,
         acc_sh, gbuf, rowv, colv, normv, invv, asub, hsub, osub,
         statb, rootv) = refs
    else:
        (h_hbm, norm_hbm, row_hbm, col_hbm, invdeg_hbm, root_hbm,
         a_hbm, c_hbm, batch_hbm,
         ssum_hbm, ssq_hbm, pooled_hbm,
         acc_sh, gbuf, rowv, colv, normv, invv, asub, hsub, osub,
         statb, rootv, av, cv, batv, poolb) = refs

    tid = lax.axis_index("s")
    cid = lax.axis_index("c")
    pltpu.sync_copy(row_hbm.at[tid], rowv)
    pltpu.sync_copy(col_hbm.at[tid], colv)
    pltpu.sync_copy(norm_hbm.at[pl.ds(tid * EPT, EPT)], normv)
    pltpu.sync_copy(invdeg_hbm.at[pl.ds(tid * SLICE, SLICE)], invv)
    if layer == 1:
        pltpu.sync_copy(batch_hbm.at[pl.ds(tid * SLICE, SLICE)], batv)

    for r in range(2):
        for j in range(8):
            statb[r, 0, pl.ds(j * 16, 16)] = jnp.zeros((16,), jnp.float32)
    if layer == 1:
        def zpool(i, _):
            poolb[pl.ds(i * 16, 16)] = jnp.zeros((16,), jnp.float32)
            return 0
        lax.fori_loop(0, (GP * F) // 16, zpool, 0)

    iota16 = lax.broadcasted_iota(jnp.int32, (16,), 0)

    def chunk_body(ci, _):
        c = 2 * ci + cid
        pltpu.sync_copy(root_hbm.at[c], rootv)
        if layer == 1:
            pltpu.sync_copy(a_hbm.at[c], av)
            pltpu.sync_copy(c_hbm.at[c], cv)

        def zg(i, _):
            for j in range(8):
                gbuf[i, pl.ds(j * 16, 16)] = jnp.zeros((16,), jnp.float32)
            return 0
        lax.fori_loop(0, 128, zg, 0)

        def zacc(s, _):
            pltpu.sync_copy(gbuf,
                            acc_sh.at[pl.ds(tid * SLICE + s * 128, 128)])
            return 0
        lax.fori_loop(0, SLICE // 128, zacc, 0)
        plsc.subcore_barrier()

        def msg(b, _):
            pltpu.sync_copy(h_hbm.at[c].at[rowv.at[b]], gbuf)

            def edge(e, _):
                nsp = plsc.load_gather(normv, [_i16(b * 128 + e)])
                for j in range(8):
                    sl = pl.ds(j * 16, 16)
                    g = gbuf[e, sl]
                    if layer == 0:
                        v = jnp.maximum(g, 0.0) * nsp
                    else:
                        u = jnp.maximum(av[0, sl] * g + cv[0, sl], 0.0)
                        v = u * nsp
                    gbuf[e, sl] = v
                return 0
            lax.fori_loop(0, 128, edge, 0)
            pltpu.sync_copy(gbuf, acc_sh.at[colv.at[b]], add=True)
            return 0
        lax.fori_loop(0, NB, msg, 0)
        plsc.subcore_barrier()

        def fin(s, _):
            r0 = tid * SLICE + s * 32
            pltpu.sync_copy(acc_sh.at[pl.ds(r0, 32)], asub)
            pltpu.sync_copy(h_hbm.at[c, pl.ds(r0, 32)], hsub)

            def rowf(i, _):
                rg = r0 + i
                validf = jnp.where(rg < N, 1.0, 0.0)
                iv = plsc.load_gather(invv, [_i16(s * 32 + i)])
                if layer == 1:
                    gid = plsc.load_gather(batv, [_i16(s * 32 + i)])
                for j in range(8):
                    sl = pl.ds(j * 16, 16)
                    hh = hsub[i, sl]
                    aa = asub[i, sl]
                    if layer == 0:
                        self_t = jnp.maximum(hh + rootv[0, sl], 0.0) * iv
                    else:
                        u = jnp.maximum(av[0, sl] * hh + cv[0, sl], 0.0)
                        self_t = jnp.maximum(u + rootv[0, sl], 0.0) * iv
                    o = (aa + self_t) * validf
                    statb[0, 0, sl] = statb[0, 0, sl] + o
                    statb[1, 0, sl] = statb[1, 0, sl] + o * o
                    if layer == 0:
                        osub[i, sl] = o
                    else:
                        plsc.addupdate_scatter(
                            poolb, [gid * F + j * 16 + iota16], o)
                return 0
            lax.fori_loop(0, 32, rowf, 0)
            if layer == 0:
                pltpu.sync_copy(osub, out_hbm.at[c, pl.ds(r0, 32)])
            return 0
        lax.fori_loop(0, SLICE // 32, fin, 0)

        pltpu.sync_copy(statb.at[0], ssum_hbm.at[c, tid])
        pltpu.sync_copy(statb.at[1], ssq_hbm.at[c, tid])
        for r in range(2):
            for j in range(8):
                statb[r, 0, pl.ds(j * 16, 16)] = jnp.zeros((16,), jnp.float32)
        if layer == 1:
            pltpu.sync_copy(
                poolb, pooled_hbm.at[pl.ds((c * NT + tid) * (GP * F),
                                           GP * F)])

            def rezpool(i, _):
                poolb[pl.ds(i * 16, 16)] = jnp.zeros((16,), jnp.float32)
                return 0
            lax.fori_loop(0, (GP * F) // 16, rezpool, 0)
        return 0
    lax.fori_loop(0, C // 2, chunk_body, 0)


_COMMON_SCRATCH = [
    pltpu.VMEM_SHARED((NP, F), jnp.float32),
    pltpu.VMEM((128, F), jnp.float32),
    pltpu.VMEM((NB, 128), jnp.int32),
    pltpu.VMEM((NB, 128), jnp.int32),
    pltpu.VMEM((EPT,), jnp.float32),
    pltpu.VMEM((SLICE,), jnp.float32),
    pltpu.VMEM((32, F), jnp.float32),
    pltpu.VMEM((32, F), jnp.float32),
    pltpu.VMEM((32, F), jnp.float32),
    pltpu.VMEM((2, 1, F), jnp.float32),
    pltpu.VMEM((1, F), jnp.float32),
]

_conv0 = pl.kernel(
    functools.partial(_conv_body, 0),
    out_type=(
        jax.ShapeDtypeStruct((C, NP, F), jnp.float32),
        jax.ShapeDtypeStruct((C, NT, 1, F), jnp.float32),
        jax.ShapeDtypeStruct((C, NT, 1, F), jnp.float32),
    ),
    mesh=_mesh,
    compiler_params=_SC_PARAMS,
    scratch_types=_COMMON_SCRATCH,
)

_conv1 = pl.kernel(
    functools.partial(_conv_body, 1),
    out_type=(
        jax.ShapeDtypeStruct((C, NT, 1, F), jnp.float32),
        jax.ShapeDtypeStruct((C, NT, 1, F), jnp.float32),
        jax.ShapeDtypeStruct((C * NT * GP * F,), jnp.float32),
    ),
    mesh=_mesh,
    compiler_params=_SC_PARAMS,
    scratch_types=_COMMON_SCRATCH + [
        pltpu.VMEM((1, F), jnp.float32),
        pltpu.VMEM((1, F), jnp.float32),
        pltpu.VMEM((SLICE,), jnp.int32),
        pltpu.VMEM((GP * F,), jnp.float32),
    ],
)


def _mm_body(x_ref, w_ref, scale_ref, bias_ref, out_ref):
    mm = lax.dot_general(x_ref[...], w_ref[...], (((1,), (1,)), ((), ())),
                         preferred_element_type=jnp.float32)
    out_ref[0] = (mm + bias_ref[0, 0][None, :]) * scale_ref[0, 0][None, :]


def _emb_matmul(x_pad, wemb_pad, scale3, bias3):
    return pl.pallas_call(
        _mm_body,
        grid=(NP // 512, C),
        in_specs=[
            pl.BlockSpec((512, IN_DIM), lambda i, c: (i, 0)),
            pl.BlockSpec((F, IN_DIM), lambda i, c: (c, 0)),
            pl.BlockSpec((1, 1, F), lambda i, c: (c, 0, 0)),
            pl.BlockSpec((1, 1, F), lambda i, c: (c, 0, 0)),
        ],
        out_specs=pl.BlockSpec((1, 512, F), lambda i, c: (c, i, 0)),
        out_shape=jax.ShapeDtypeStruct((C, NP, F), jnp.float32),
    )(x_pad, wemb_pad, scale3, bias3)


def _bn_body(ssum_ref, ssq_ref, gamma_ref, beta_ref, a_ref, c_ref):
    s = jnp.sum(ssum_ref[:, :, 0, :], axis=1)
    q = jnp.sum(ssq_ref[:, :, 0, :], axis=1)
    mu = s / float(N)
    var = q / float(N) - mu * mu
    a = gamma_ref[:, 0, :] * lax.rsqrt(var + 1e-5)
    cc = beta_ref[:, 0, :] - mu * a
    a_ref[...] = a[:, None, :]
    c_ref[...] = cc[:, None, :]


def _bn_affine(ssum, ssq, gamma3, beta3):
    return pl.pallas_call(
        _bn_body,
        out_shape=(
            jax.ShapeDtypeStruct((C, 1, F), jnp.float32),
            jax.ShapeDtypeStruct((C, 1, F), jnp.float32),
        ),
    )(ssum, ssq, gamma3, beta3)


def _head_body(pooled_ref, ssum_ref, ssq_ref, gamma_ref, beta_ref,
               batch_ref, wp_ref, bp_ref, out_ref):
    b2 = batch_ref[...]
    iota_g = lax.broadcasted_iota(jnp.int32, (NUM_GRAPHS, NP // 128, 128), 0)
    eq = (b2[None, :, :] == iota_g).astype(jnp.float32)
    cnt = jnp.sum(eq, axis=(1, 2))[:, None]
    acc = jnp.zeros((NUM_GRAPHS, F), jnp.float32)
    for c in range(C):
        psum = jnp.sum(pooled_ref[c], axis=0)[:NUM_GRAPHS]
        s = jnp.sum(ssum_ref[c, :, 0, :], axis=0)
        q = jnp.sum(ssq_ref[c, :, 0, :], axis=0)
        mu = s / float(N)
        var = q / float(N) - mu * mu
        a2 = gamma_ref[c, 0] * lax.rsqrt(var + 1e-5)
        c2 = beta_ref[c, 0] - mu * a2
        pb = psum * a2[None, :] + cnt * c2[None, :]
        acc = acc + lax.dot_general(pb, wp_ref[c], (((1,), (1,)), ((), ())),
                                    preferred_element_type=jnp.float32)
    out_ref[...] = acc + bp_ref[0, 0][None, :]


def _head(pooled4, ssum, ssq, gamma3, beta3, batch2d, wp_chunks, bp3):
    return pl.pallas_call(
        _head_body,
        out_shape=jax.ShapeDtypeStruct((NUM_GRAPHS, F), jnp.float32),
    )(pooled4, ssum, ssq, gamma3, beta3, batch2d, wp_chunks, bp3)


def _topk_body(w_ref, out_ref):
    w = w_ref[...]
    iota_lin = (lax.broadcasted_iota(jnp.int32, (C, F), 0) * F +
                lax.broadcasted_iota(jnp.int32, (C, F), 1))
    oidx = lax.broadcasted_iota(jnp.int32, (8, 128), 0) * 128 + \
        lax.broadcasted_iota(jnp.int32, (8, 128), 1)
    out = jnp.zeros((8, 128), jnp.int32)
    for k in range(K):
        m = jnp.max(w)
        idx = jnp.min(jnp.where(w == m, iota_lin, jnp.int32(2**30)))
        out = jnp.where(oidx == k, idx, out)
        w = jnp.where(iota_lin == idx, -jnp.inf, w)
    out_ref[...] = out


def _topk(w_pad2):
    return pl.pallas_call(
        _topk_body,
        out_shape=jax.ShapeDtypeStruct((8, 128), jnp.int32),
    )(w_pad2)


def kernel(x, edge_index, batch, W, W_emb, b_emb, root0, root1,
           gamma0, beta0, gamma1, beta1, Wp, bp):
    f32 = jnp.float32
    x_pad = jnp.pad(x, ((0, NP - N), (0, 0)))
    wemb_pad = jnp.pad(W_emb, ((0, EMB_P - EMB), (0, 0)))
    scale3 = jnp.pad(W, (0, EMB_P - EMB)).reshape(C, 1, F)
    bias3 = jnp.pad(b_emb, (0, EMB_P - EMB)).reshape(C, 1, F)
    root0c = jnp.pad(root0, (0, EMB_P - EMB)).reshape(C, 1, F)
    root1c = jnp.pad(root1, (0, EMB_P - EMB)).reshape(C, 1, F)
    g0 = jnp.pad(gamma0, (0, EMB_P - EMB)).reshape(C, 1, F)
    b0 = jnp.pad(beta0, (0, EMB_P - EMB)).reshape(C, 1, F)
    g1 = jnp.pad(gamma1, (0, EMB_P - EMB)).reshape(C, 1, F)
    b1 = jnp.pad(beta1, (0, EMB_P - EMB)).reshape(C, 1, F)

    npad = EP - E
    row_p = jnp.concatenate([edge_index[0], jnp.zeros((npad,), jnp.int32)])
    col_p = jnp.concatenate(
        [edge_index[1], N + (jnp.arange(npad, dtype=jnp.int32) % (NP - N))])
    row3d = row_p.reshape(NT, NB, 128)
    col3d = col_p.reshape(NT, NB, 128)

    batch_pad = jnp.concatenate(
        [batch, jnp.full((NP - N,), NUM_GRAPHS, jnp.int32)])
    batch2d = batch_pad.reshape(NP // 128, 128)

    ar = jnp.arange(TAB, dtype=f32)
    dis_tab = (ar + (ar == 0)) ** -0.5

    wp_pad = jnp.pad(Wp, ((0, F - OUT), (0, EMB_P - EMB)))
    wp_chunks = jnp.transpose(wp_pad.reshape(F, C, F), (1, 0, 2))
    bp3 = jnp.pad(bp, (0, F - OUT)).reshape(1, 1, F)

    w_pad2 = jnp.pad(W, (0, EMB_P - EMB),
                     constant_values=-jnp.inf).reshape(C, F)

    norm4d, invdeg = _k1(row3d, col3d, dis_tab)
    h0 = _emb_matmul(x_pad, wemb_pad, scale3, bias3)
    out1, s1, q1 = _conv0(h0, norm4d, row3d, col3d, invdeg, root0c)
    a1, c1 = _bn_affine(s1, q1, g0, b0)
    s2, q2, pooled = _conv1(out1, norm4d, row3d, col3d, invdeg, root1c,
                            a1, c1, batch_pad)
    pooled4 = pooled.reshape(C, NT, GP, F)
    pred_pad = _head(pooled4, s2, q2, g1, b1, batch2d, wp_chunks, bp3)
    topk_pad = _topk(w_pad2)

    pred = pred_pad[:, :OUT]
    topk_indices = topk_pad.reshape(-1)[:K]
    return pred, topk_indices


# speedup vs baseline: 1.5901x; 1.5901x over previous
import jax
import jax.numpy as jnp
from jax import lax
from jax.experimental import pallas as pl
from jax.experimental.pallas import tpu as pltpu
from jax.experimental.pallas import tpu_sc as plsc

N = 10000
E = 40000
IN_DIM = 512
EMB = 3000
NUM_GRAPHS = 16
K = 50
OUT = 33

NP = 10240
F = 128
C = 24
EMB_P = C * F
NC = 2
NT = 16
EPT = 2560
EP = NT * EPT
EB = 64
NBM = EPT // EB
SLICE = NP // NT
TAB = 512

_mesh = plsc.VectorSubcoreMesh(core_axis_name="c", subcore_axis_name="s")
_SC_PARAMS = pltpu.CompilerParams(needs_layout_passes=False)


def _i16(v):
    return jnp.full((16,), v, dtype=jnp.int32)


def _k1_body(row_hbm, col_hbm, tab_hbm, norm_hbm, invdeg_hbm,
             acc_sh, rowv, colv, tabv, degv, disv, normb, onesv, zerosv, invb):
    tid = lax.axis_index("s")
    cid = lax.axis_index("c")
    pltpu.sync_copy(row_hbm.at[tid], rowv)
    pltpu.sync_copy(col_hbm.at[tid], colv)
    pltpu.sync_copy(tab_hbm, tabv)

    def fill(i, _):
        onesv[pl.ds(i * 16, 16)] = jnp.full((16,), 1.0, jnp.float32)
        return 0
    lax.fori_loop(0, EB // 16, fill, 0)

    def fillz(i, _):
        zerosv[pl.ds(i * 16, 16)] = jnp.zeros((16,), jnp.float32)
        return 0
    lax.fori_loop(0, SLICE // 16, fillz, 0)

    pltpu.sync_copy(zerosv, acc_sh.at[pl.ds(tid * SLICE, SLICE)])
    plsc.subcore_barrier()

    def cnt(b, _):
        pltpu.sync_copy(onesv, acc_sh.at[colv.at[b]], add=True)
        return 0
    lax.fori_loop(0, NBM, cnt, 0)
    plsc.subcore_barrier()

    pltpu.sync_copy(acc_sh, degv)

    def mkdis(i, _):
        d = degv[pl.ds(i * 16, 16)] + 1.0
        di = jnp.minimum(d.astype(jnp.int32), TAB - 1)
        disv[pl.ds(i * 16, 16)] = plsc.load_gather(tabv, [di])
        return 0
    lax.fori_loop(0, NP // 16, mkdis, 0)

    def mknorm(bi, _):
        b = cid * (NBM // 2) + bi
        for g in range(EB // 16):
            sl = pl.ds(g * 16, 16)
            r16 = rowv[b, sl]
            c16 = colv[b, sl]
            normb[pl.ds(bi * EB + g * 16, 16)] = (
                plsc.load_gather(disv, [r16]) *
                plsc.load_gather(disv, [c16]))
        return 0
    lax.fori_loop(0, NBM // 2, mknorm, 0)
    pltpu.sync_copy(normb,
                    norm_hbm.at[pl.ds(tid * EPT + cid * (EPT // 2),
                                      EPT // 2)])

    @pl.when(cid == 0)
    def _():
        def mkinv(i, _):
            d = degv[pl.ds(tid * SLICE + i * 16, 16)] + 1.0
            invb[pl.ds(i * 16, 16)] = 1.0 / d
            return 0
        lax.fori_loop(0, SLICE // 16, mkinv, 0)
        pltpu.sync_copy(invb, invdeg_hbm.at[pl.ds(tid * SLICE, SLICE)])


_k1 = pl.kernel(
    _k1_body,
    out_type=(
        jax.ShapeDtypeStruct((EP,), jnp.float32),
        jax.ShapeDtypeStruct((NP,), jnp.float32),
    ),
    mesh=_mesh,
    compiler_params=_SC_PARAMS,
    scratch_types=[
        pltpu.VMEM_SHARED((NP,), jnp.float32),
        pltpu.VMEM((NBM, EB), jnp.int32),
        pltpu.VMEM((NBM, EB), jnp.int32),
        pltpu.VMEM((TAB,), jnp.float32),
        pltpu.VMEM((NP,), jnp.float32),
        pltpu.VMEM((NP,), jnp.float32),
        pltpu.VMEM((EPT // 2,), jnp.float32),
        pltpu.VMEM((EB,), jnp.float32),
        pltpu.VMEM((SLICE,), jnp.float32),
        pltpu.VMEM((SLICE,), jnp.float32),
    ],
)


def _conv_body(tbl_hbm, hs_hbm, norm_hbm, row_hbm, col_hbm, invdeg_hbm,
               root_hbm, out_hbm,
               acc_sh, g0, g1, rowv, colv, normv, invv, asub, hsub, rootv,
               sem0, sem1):
    tid = lax.axis_index("s")
    cid = lax.axis_index("c")
    pltpu.sync_copy(row_hbm.at[tid], rowv)
    pltpu.sync_copy(col_hbm.at[tid], colv)
    pltpu.sync_copy(norm_hbm.at[pl.ds(tid * EPT, EPT)], normv)
    pltpu.sync_copy(invdeg_hbm.at[pl.ds(tid * SLICE, SLICE)], invv)

    def chunk_body(ci, _):
        c = 2 * ci + cid
        pltpu.sync_copy(root_hbm.at[c], rootv)

        def zg(i, _):
            for j in range(8):
                g0[i, pl.ds(j * 16, 16)] = jnp.zeros((16,), jnp.float32)
            return 0
        lax.fori_loop(0, EB, zg, 0)

        def zacc(s, _):
            pltpu.sync_copy(g0, acc_sh.at[pl.ds(tid * SLICE + s * EB, EB)])
            return 0
        lax.fori_loop(0, SLICE // EB, zacc, 0)
        plsc.subcore_barrier()

        def gath(b, buf, sem):
            return pltpu.async_copy(tbl_hbm.at[c].at[rowv.at[b]], buf, sem)

        def wait(b, buf, sem):
            pltpu.make_async_copy(tbl_hbm.at[c].at[rowv.at[b]], buf,
                                  sem).wait()

        def xform(b, buf):
            def edge(e, _):
                nsp = plsc.load_gather(normv, [_i16(b * EB + e)])
                for j in range(8):
                    sl = pl.ds(j * 16, 16)
                    buf[e, sl] = buf[e, sl] * nsp
                return 0
            lax.fori_loop(0, EB, edge, 0)

        gath(0, g0, sem0)

        def msg(bb, _):
            b0 = 2 * bb
            b1 = 2 * bb + 1
            gath(b1, g1, sem1)
            wait(b0, g0, sem0)
            xform(b0, g0)
            pltpu.sync_copy(g0, acc_sh.at[colv.at[b0]], add=True)

            @pl.when(bb + 1 < NBM // 2)
            def _():
                gath(b0 + 2, g0, sem0)
            wait(b1, g1, sem1)
            xform(b1, g1)
            pltpu.sync_copy(g1, acc_sh.at[colv.at[b1]], add=True)
            return 0
        lax.fori_loop(0, NBM // 2, msg, 0)
        plsc.subcore_barrier()

        def fin(s, _):
            r0 = tid * SLICE + s * 32
            pltpu.sync_copy(acc_sh.at[pl.ds(r0, 32)], asub)
            pltpu.sync_copy(hs_hbm.at[c, pl.ds(r0, 32)], hsub)

            def rowf(i, _):
                rg = r0 + i
                validf = jnp.where(rg < N, 1.0, 0.0)
                iv = plsc.load_gather(invv, [_i16(s * 32 + i)]) * validf
                for j in range(8):
                    sl = pl.ds(j * 16, 16)
                    self_t = jnp.maximum(hsub[i, sl] + rootv[0, sl], 0.0)
                    asub[i, sl] = asub[i, sl] * validf + self_t * iv
                return 0
            lax.fori_loop(0, 32, rowf, 0)
            pltpu.sync_copy(asub, out_hbm.at[c, pl.ds(r0, 32)])
            return 0
        lax.fori_loop(0, SLICE // 32, fin, 0)
        return 0
    lax.fori_loop(0, C // 2, chunk_body, 0)


_conv = pl.kernel(
    _conv_body,
    out_type=jax.ShapeDtypeStruct((C, NP, F), jnp.float32),
    mesh=_mesh,
    compiler_params=_SC_PARAMS,
    scratch_types=[
        pltpu.VMEM_SHARED((NP, F), jnp.float32),
        pltpu.VMEM((EB, F), jnp.float32),
        pltpu.VMEM((EB, F), jnp.float32),
        pltpu.VMEM((NBM, EB), jnp.int32),
        pltpu.VMEM((NBM, EB), jnp.int32),
        pltpu.VMEM((EPT,), jnp.float32),
        pltpu.VMEM((SLICE,), jnp.float32),
        pltpu.VMEM((32, F), jnp.float32),
        pltpu.VMEM((32, F), jnp.float32),
        pltpu.VMEM((1, F), jnp.float32),
        pltpu.SemaphoreType.DMA,
        pltpu.SemaphoreType.DMA,
    ],
)


def _mm_body(x_ref, w_ref, scale_ref, bias_ref, h_ref, t_ref):
    mm = lax.dot_general(x_ref[...], w_ref[...], (((1,), (1,)), ((), ())),
                         preferred_element_type=jnp.float32)
    h = (mm + bias_ref[0, 0][None, :]) * scale_ref[0, 0][None, :]
    h_ref[0] = h
    t_ref[0] = jnp.maximum(h, 0.0)


def _emb_matmul(x_pad, wemb_pad, scale3, bias3):
    return pl.pallas_call(
        _mm_body,
        grid=(NP // 512, C),
        in_specs=[
            pl.BlockSpec((512, IN_DIM), lambda i, c: (i, 0)),
            pl.BlockSpec((F, IN_DIM), lambda i, c: (c, 0)),
            pl.BlockSpec((1, 1, F), lambda i, c: (c, 0, 0)),
            pl.BlockSpec((1, 1, F), lambda i, c: (c, 0, 0)),
        ],
        out_specs=[
            pl.BlockSpec((1, 512, F), lambda i, c: (c, i, 0)),
            pl.BlockSpec((1, 512, F), lambda i, c: (c, i, 0)),
        ],
        out_shape=(
            jax.ShapeDtypeStruct((C, NP, F), jnp.float32),
            jax.ShapeDtypeStruct((C, NP, F), jnp.float32),
        ),
    )(x_pad, wemb_pad, scale3, bias3)


def _stats_body(h_ref, gamma_ref, beta_ref, a_ref, c_ref):
    d = h_ref[0]
    s = jnp.sum(d, axis=0)
    q = jnp.sum(d * d, axis=0)
    mu = s / float(N)
    var = q / float(N) - mu * mu
    a = gamma_ref[0, 0] * lax.rsqrt(var + 1e-5)
    cc = beta_ref[0, 0] - mu * a
    a_ref[0, 0] = a
    c_ref[0, 0] = cc


def _stats_affine(h, gamma3, beta3):
    return pl.pallas_call(
        _stats_body,
        grid=(C,),
        in_specs=[
            pl.BlockSpec((1, NP, F), lambda c: (c, 0, 0)),
            pl.BlockSpec((1, 1, F), lambda c: (c, 0, 0)),
            pl.BlockSpec((1, 1, F), lambda c: (c, 0, 0)),
        ],
        out_specs=[
            pl.BlockSpec((1, 1, F), lambda c: (c, 0, 0)),
            pl.BlockSpec((1, 1, F), lambda c: (c, 0, 0)),
        ],
        out_shape=(
            jax.ShapeDtypeStruct((C, 1, F), jnp.float32),
            jax.ShapeDtypeStruct((C, 1, F), jnp.float32),
        ),
    )(h, gamma3, beta3)


def _t1_body(h_ref, a_ref, c_ref, t_ref):
    t_ref[0] = jnp.maximum(h_ref[0] * a_ref[0, 0][None, :] +
                           c_ref[0, 0][None, :], 0.0)


def _t1(h, a3, c3):
    return pl.pallas_call(
        _t1_body,
        grid=(C, NP // 1024),
        in_specs=[
            pl.BlockSpec((1, 1024, F), lambda c, i: (c, i, 0)),
            pl.BlockSpec((1, 1, F), lambda c, i: (c, 0, 0)),
            pl.BlockSpec((1, 1, F), lambda c, i: (c, 0, 0)),
        ],
        out_specs=pl.BlockSpec((1, 1024, F), lambda c, i: (c, i, 0)),
        out_shape=jax.ShapeDtypeStruct((C, NP, F), jnp.float32),
    )(h, a3, c3)


def _head_body(h_ref, gamma_ref, beta_ref, batch_ref, wp_ref, bp_ref,
               out_ref):
    c = pl.program_id(0)
    d = h_ref[0]
    s = jnp.sum(d, axis=0)
    q = jnp.sum(d * d, axis=0)
    mu = s / float(N)
    var = q / float(N) - mu * mu
    a2 = gamma_ref[0, 0] * lax.rsqrt(var + 1e-5)
    c2 = beta_ref[0, 0] - mu * a2
    iota_g = lax.broadcasted_iota(jnp.int32, (NUM_GRAPHS, NP), 0)
    oh = (batch_ref[0, :][None, :] == iota_g).astype(jnp.float32)
    psum = lax.dot_general(oh, d, (((1,), (0,)), ((), ())),
                           preferred_element_type=jnp.float32)
    cnt = jnp.sum(oh, axis=1)[:, None]
    pb = psum * a2[None, :] + cnt * c2[None, :]
    pr = lax.dot_general(pb, wp_ref[0], (((1,), (1,)), ((), ())),
                         preferred_element_type=jnp.float32)

    @pl.when(c == 0)
    def _():
        out_ref[...] = jnp.broadcast_to(bp_ref[0, 0][None, :],
                                        (NUM_GRAPHS, F))
    out_ref[...] += pr


def _head(h2, gamma3, beta3, batch2d, wp_chunks, bp3):
    return pl.pallas_call(
        _head_body,
        grid=(C,),
        in_specs=[
            pl.BlockSpec((1, NP, F), lambda c: (c, 0, 0)),
            pl.BlockSpec((1, 1, F), lambda c: (c, 0, 0)),
            pl.BlockSpec((1, 1, F), lambda c: (c, 0, 0)),
            pl.BlockSpec((8, NP), lambda c: (0, 0)),
            pl.BlockSpec((1, F, F), lambda c: (c, 0, 0)),
            pl.BlockSpec((1, 1, F), lambda c: (0, 0, 0)),
        ],
        out_specs=pl.BlockSpec((NUM_GRAPHS, F), lambda c: (0, 0)),
        out_shape=jax.ShapeDtypeStruct((NUM_GRAPHS, F), jnp.float32),
    )(h2, gamma3, beta3, batch2d, wp_chunks, bp3)


def _topk_body(w_ref, out_ref):
    w = w_ref[...]
    iota_lin = (lax.broadcasted_iota(jnp.int32, (C, F), 0) * F +
                lax.broadcasted_iota(jnp.int32, (C, F), 1))
    oidx = lax.broadcasted_iota(jnp.int32, (8, 128), 0) * 128 + \
        lax.broadcasted_iota(jnp.int32, (8, 128), 1)
    out = jnp.zeros((8, 128), jnp.int32)
    for k in range(K):
        m = jnp.max(w)
        idx = jnp.min(jnp.where(w == m, iota_lin, jnp.int32(2**30)))
        out = jnp.where(oidx == k, idx, out)
        w = jnp.where(iota_lin == idx, -jnp.inf, w)
    out_ref[...] = out


def _topk(w_pad2):
    return pl.pallas_call(
        _topk_body,
        out_shape=jax.ShapeDtypeStruct((8, 128), jnp.int32),
    )(w_pad2)


def kernel(x, edge_index, batch, W, W_emb, b_emb, root0, root1,
           gamma0, beta0, gamma1, beta1, Wp, bp):
    f32 = jnp.float32
    x_pad = jnp.pad(x, ((0, NP - N), (0, 0)))
    wemb_pad = jnp.pad(W_emb, ((0, EMB_P - EMB), (0, 0)))
    scale3 = jnp.pad(W, (0, EMB_P - EMB)).reshape(C, 1, F)
    bias3 = jnp.pad(b_emb, (0, EMB_P - EMB)).reshape(C, 1, F)
    root0c = jnp.pad(root0, (0, EMB_P - EMB)).reshape(C, 1, F)
    root1c = jnp.pad(root1, (0, EMB_P - EMB)).reshape(C, 1, F)
    g0 = jnp.pad(gamma0, (0, EMB_P - EMB)).reshape(C, 1, F)
    b0 = jnp.pad(beta0, (0, EMB_P - EMB)).reshape(C, 1, F)
    g1 = jnp.pad(gamma1, (0, EMB_P - EMB)).reshape(C, 1, F)
    b1 = jnp.pad(beta1, (0, EMB_P - EMB)).reshape(C, 1, F)

    npad = EP - E
    row_p = jnp.concatenate([edge_index[0], jnp.zeros((npad,), jnp.int32)])
    col_p = jnp.concatenate(
        [edge_index[1], N + (jnp.arange(npad, dtype=jnp.int32) % (NP - N))])
    row3d = row_p.reshape(NT, NBM, EB)
    col3d = col_p.reshape(NT, NBM, EB)

    batch_pad = jnp.concatenate(
        [batch, jnp.full((NP - N,), NUM_GRAPHS, jnp.int32)])
    batch2d = jnp.broadcast_to(batch_pad[None, :], (8, NP))

    ar = jnp.arange(TAB, dtype=f32)
    dis_tab = (ar + (ar == 0)) ** -0.5

    wp_pad = jnp.pad(Wp, ((0, F - OUT), (0, EMB_P - EMB)))
    wp_chunks = jnp.transpose(wp_pad.reshape(F, C, F), (1, 0, 2))
    bp3 = jnp.pad(bp, (0, F - OUT)).reshape(1, 1, F)

    w_pad2 = jnp.pad(W, (0, EMB_P - EMB),
                     constant_values=-jnp.inf).reshape(C, F)

    norm, invdeg = _k1(row3d, col3d, dis_tab)
    h0, t0 = _emb_matmul(x_pad, wemb_pad, scale3, bias3)
    out1 = _conv(t0, h0, norm, row3d, col3d, invdeg, root0c)
    a1, c1 = _stats_affine(out1, g0, b0)
    t1 = _t1(out1, a1, c1)
    out2 = _conv(t1, t1, norm, row3d, col3d, invdeg, root1c)
    pred_pad = _head(out2, g1, b1, batch2d, wp_chunks, bp3)
    topk_pad = _topk(w_pad2)

    pred = pred_pad[:, :OUT]
    topk_indices = topk_pad.reshape(-1)[:K]
    return pred, topk_indices


# speedup vs baseline: 1.7963x; 1.1297x over previous
import jax
import jax.numpy as jnp
from jax import lax
from jax.experimental import pallas as pl
from jax.experimental.pallas import tpu as pltpu
from jax.experimental.pallas import tpu_sc as plsc

N = 10000
E = 40000
IN_DIM = 512
EMB = 3000
NUM_GRAPHS = 16
K = 50
OUT = 33

NP = 10240
F = 128
C = 24
EMB_P = C * F
NC = 2
NT = 16
EPT = 2560
EP = NT * EPT
EB = 64
NBM = EPT // EB
SLICE = NP // NT

_mesh = plsc.VectorSubcoreMesh(core_axis_name="c", subcore_axis_name="s")
_SC_PARAMS = pltpu.CompilerParams(needs_layout_passes=False)


def _i16(v):
    return jnp.full((16,), v, dtype=jnp.int32)


def _k1_body(col_hbm, deg_hbm, acc_sh, colv, onesv, zerosv, degb):
    tid = lax.axis_index("s")
    cid = lax.axis_index("c")
    pltpu.sync_copy(col_hbm.at[tid], colv)

    def fill(i, _):
        onesv[pl.ds(i * 16, 16)] = jnp.full((16,), 1.0, jnp.float32)
        return 0
    lax.fori_loop(0, EB // 16, fill, 0)

    def fillz(i, _):
        zerosv[pl.ds(i * 16, 16)] = jnp.zeros((16,), jnp.float32)
        return 0
    lax.fori_loop(0, SLICE // 16, fillz, 0)

    pltpu.sync_copy(zerosv, acc_sh.at[pl.ds(tid * SLICE, SLICE)])
    plsc.subcore_barrier()

    def cnt(b, _):
        pltpu.sync_copy(onesv, acc_sh.at[colv.at[b]], add=True)
        return 0
    lax.fori_loop(0, NBM, cnt, 0)
    plsc.subcore_barrier()

    @pl.when(cid == 0)
    def _():
        pltpu.sync_copy(acc_sh.at[pl.ds(tid * SLICE, SLICE)], degb)

        def mkdeg(i, _):
            degb[pl.ds(i * 16, 16)] = degb[pl.ds(i * 16, 16)] + 1.0
            return 0
        lax.fori_loop(0, SLICE // 16, mkdeg, 0)
        pltpu.sync_copy(degb, deg_hbm.at[pl.ds(tid * SLICE, SLICE)])


_k1 = pl.kernel(
    _k1_body,
    out_type=jax.ShapeDtypeStruct((NP,), jnp.float32),
    mesh=_mesh,
    compiler_params=_SC_PARAMS,
    scratch_types=[
        pltpu.VMEM_SHARED((NP,), jnp.float32),
        pltpu.VMEM((NBM, EB), jnp.int32),
        pltpu.VMEM((EB,), jnp.float32),
        pltpu.VMEM((SLICE,), jnp.float32),
        pltpu.VMEM((SLICE,), jnp.float32),
    ],
)


def _conv_body(tbl_hbm, hs_hbm, row_hbm, col_hbm, inv_hbm, s_hbm, dis_hbm,
               root_hbm, out_hbm,
               acc_sh, g0, g1, rowv, colv, invv, sv, disv, a0, h0b, a1, h1b,
               rootv, sem0, sem1, sf0, sf1):
    tid = lax.axis_index("s")
    cid = lax.axis_index("c")
    pltpu.sync_copy(row_hbm.at[tid], rowv)
    pltpu.sync_copy(col_hbm.at[tid], colv)
    pltpu.sync_copy(inv_hbm.at[pl.ds(tid * SLICE, SLICE)], invv)
    pltpu.sync_copy(s_hbm.at[pl.ds(tid * SLICE, SLICE)], sv)
    pltpu.sync_copy(dis_hbm.at[pl.ds(tid * SLICE, SLICE)], disv)

    def zg(i, _):
        for j in range(8):
            g0[i, pl.ds(j * 16, 16)] = jnp.zeros((16,), jnp.float32)
        return 0
    lax.fori_loop(0, EB, zg, 0)

    def zacc(s, _):
        pltpu.sync_copy(g0, acc_sh.at[pl.ds(tid * SLICE + s * EB, EB)])
        return 0
    lax.fori_loop(0, SLICE // EB, zacc, 0)

    def chunk_body(ci, _):
        c = 2 * ci + cid
        pltpu.sync_copy(root_hbm.at[c], rootv)
        plsc.subcore_barrier()

        def gath(b, buf, sem):
            return pltpu.async_copy(tbl_hbm.at[c].at[rowv.at[b]], buf, sem)

        def wait(b, buf, sem):
            pltpu.make_async_copy(tbl_hbm.at[c].at[rowv.at[b]], buf,
                                  sem).wait()

        gath(0, g0, sem0)

        def msg(bb, _):
            b0 = 2 * bb
            b1 = 2 * bb + 1
            gath(b1, g1, sem1)
            wait(b0, g0, sem0)
            pltpu.sync_copy(g0, acc_sh.at[colv.at[b0]], add=True)

            @pl.when(bb + 1 < NBM // 2)
            def _():
                gath(b0 + 2, g0, sem0)
            wait(b1, g1, sem1)
            pltpu.sync_copy(g1, acc_sh.at[colv.at[b1]], add=True)
            return 0
        lax.fori_loop(0, NBM // 2, msg, 0)
        plsc.subcore_barrier()

        def fload(s, ab, hb, sfa, sfh):
            r0 = tid * SLICE + s * 32
            pltpu.async_copy(acc_sh.at[pl.ds(r0, 32)], ab, sfa)
            pltpu.async_copy(hs_hbm.at[c, pl.ds(r0, 32)], hb, sfh)

        def fwait(s, ab, hb, sfa, sfh):
            r0 = tid * SLICE + s * 32
            pltpu.make_async_copy(acc_sh.at[pl.ds(r0, 32)], ab, sfa).wait()
            pltpu.make_async_copy(hs_hbm.at[c, pl.ds(r0, 32)], hb,
                                  sfh).wait()

        def fcomp(s, ab, hb):
            r0 = tid * SLICE + s * 32

            def rowf(i, _):
                rg = r0 + i
                validf = jnp.where(rg < N, 1.0, 0.0)
                iv = plsc.load_gather(invv, [_i16(s * 32 + i)]) * validf
                dd = plsc.load_gather(disv, [_i16(s * 32 + i)]) * validf
                ss = plsc.load_gather(sv, [_i16(s * 32 + i)])
                for j in range(8):
                    sl = pl.ds(j * 16, 16)
                    self_t = jnp.maximum(hb[i, sl] * ss + rootv[0, sl], 0.0)
                    ab[i, sl] = ab[i, sl] * dd + self_t * iv
                return 0
            lax.fori_loop(0, 32, rowf, 0)
            pltpu.sync_copy(ab, out_hbm.at[c, pl.ds(r0, 32)])
            def rez(i, _):
                for j in range(8):
                    ab[i, pl.ds(j * 16, 16)] = jnp.zeros((16,), jnp.float32)
                return 0
            lax.fori_loop(0, 32, rez, 0)
            pltpu.sync_copy(ab, acc_sh.at[pl.ds(r0, 32)])

        fload(0, a0, h0b, sf0, sem0)

        def fin(ss2, _):
            s0 = 2 * ss2
            s1 = 2 * ss2 + 1
            fload(s1, a1, h1b, sf1, sem1)
            fwait(s0, a0, h0b, sf0, sem0)
            fcomp(s0, a0, h0b)

            @pl.when(ss2 + 1 < SLICE // 64)
            def _():
                fload(s0 + 2, a0, h0b, sf0, sem0)
            fwait(s1, a1, h1b, sf1, sem1)
            fcomp(s1, a1, h1b)
            return 0
        lax.fori_loop(0, SLICE // 64, fin, 0)
        return 0
    lax.fori_loop(0, C // 2, chunk_body, 0)


_conv = pl.kernel(
    _conv_body,
    out_type=jax.ShapeDtypeStruct((C, NP, F), jnp.float32),
    mesh=_mesh,
    compiler_params=_SC_PARAMS,
    scratch_types=[
        pltpu.VMEM_SHARED((NP, F), jnp.float32),
        pltpu.VMEM((EB, F), jnp.float32),
        pltpu.VMEM((EB, F), jnp.float32),
        pltpu.VMEM((NBM, EB), jnp.int32),
        pltpu.VMEM((NBM, EB), jnp.int32),
        pltpu.VMEM((SLICE,), jnp.float32),
        pltpu.VMEM((SLICE,), jnp.float32),
        pltpu.VMEM((SLICE,), jnp.float32),
        pltpu.VMEM((32, F), jnp.float32),
        pltpu.VMEM((32, F), jnp.float32),
        pltpu.VMEM((32, F), jnp.float32),
        pltpu.VMEM((32, F), jnp.float32),
        pltpu.VMEM((1, F), jnp.float32),
        pltpu.SemaphoreType.DMA,
        pltpu.SemaphoreType.DMA,
        pltpu.SemaphoreType.DMA,
        pltpu.SemaphoreType.DMA,
    ],
)


def _degpost_body(deg_ref, inv_ref, s_ref, dis_ref):
    d = deg_ref[...]
    inv_ref[...] = 1.0 / d
    s_ref[...] = jnp.sqrt(d)
    dis_ref[...] = lax.rsqrt(d)


def _degpost(deg2d):
    return pl.pallas_call(
        _degpost_body,
        out_shape=(
            jax.ShapeDtypeStruct((NP // 128, 128), jnp.float32),
            jax.ShapeDtypeStruct((NP // 128, 128), jnp.float32),
            jax.ShapeDtypeStruct((NP // 128, 128), jnp.float32),
        ),
    )(deg2d)


def _mm_body(x_ref, w_ref, scale_ref, bias_ref, disb_ref, h_ref, t_ref):
    mm = lax.dot_general(x_ref[...], w_ref[...], (((1,), (1,)), ((), ())),
                         preferred_element_type=jnp.float32)
    h = (mm + bias_ref[0, 0][None, :]) * scale_ref[0, 0][None, :]
    hp = h * disb_ref[...]
    h_ref[0] = hp
    t_ref[0] = jnp.maximum(hp, 0.0)


def _emb_matmul(x_pad, wemb_pad, scale3, bias3, disb):
    return pl.pallas_call(
        _mm_body,
        grid=(NP // 512, C),
        in_specs=[
            pl.BlockSpec((512, IN_DIM), lambda i, c: (i, 0)),
            pl.BlockSpec((F, IN_DIM), lambda i, c: (c, 0)),
            pl.BlockSpec((1, 1, F), lambda i, c: (c, 0, 0)),
            pl.BlockSpec((1, 1, F), lambda i, c: (c, 0, 0)),
            pl.BlockSpec((512, F), lambda i, c: (i, 0)),
        ],
        out_specs=[
            pl.BlockSpec((1, 512, F), lambda i, c: (c, i, 0)),
            pl.BlockSpec((1, 512, F), lambda i, c: (c, i, 0)),
        ],
        out_shape=(
            jax.ShapeDtypeStruct((C, NP, F), jnp.float32),
            jax.ShapeDtypeStruct((C, NP, F), jnp.float32),
        ),
    )(x_pad, wemb_pad, scale3, bias3, disb)


def _stats_body(h_ref, gamma_ref, beta_ref, a_ref, c_ref):
    d = h_ref[0]
    s = jnp.sum(d, axis=0)
    q = jnp.sum(d * d, axis=0)
    mu = s / float(N)
    var = q / float(N) - mu * mu
    a = gamma_ref[0, 0] * lax.rsqrt(var + 1e-5)
    cc = beta_ref[0, 0] - mu * a
    a_ref[0, 0] = a
    c_ref[0, 0] = cc


def _stats_affine(h, gamma3, beta3):
    return pl.pallas_call(
        _stats_body,
        grid=(C,),
        in_specs=[
            pl.BlockSpec((1, NP, F), lambda c: (c, 0, 0)),
            pl.BlockSpec((1, 1, F), lambda c: (c, 0, 0)),
            pl.BlockSpec((1, 1, F), lambda c: (c, 0, 0)),
        ],
        out_specs=[
            pl.BlockSpec((1, 1, F), lambda c: (c, 0, 0)),
            pl.BlockSpec((1, 1, F), lambda c: (c, 0, 0)),
        ],
        out_shape=(
            jax.ShapeDtypeStruct((C, 1, F), jnp.float32),
            jax.ShapeDtypeStruct((C, 1, F), jnp.float32),
        ),
    )(h, gamma3, beta3)


def _t1_body(h_ref, a_ref, c_ref, disb_ref, t_ref):
    t_ref[0] = jnp.maximum(h_ref[0] * a_ref[0, 0][None, :] +
                           c_ref[0, 0][None, :], 0.0) * disb_ref[...]


def _t1(h, a3, c3, disb):
    return pl.pallas_call(
        _t1_body,
        grid=(C, NP // 1024),
        in_specs=[
            pl.BlockSpec((1, 1024, F), lambda c, i: (c, i, 0)),
            pl.BlockSpec((1, 1, F), lambda c, i: (c, 0, 0)),
            pl.BlockSpec((1, 1, F), lambda c, i: (c, 0, 0)),
            pl.BlockSpec((1024, F), lambda c, i: (i, 0)),
        ],
        out_specs=pl.BlockSpec((1, 1024, F), lambda c, i: (c, i, 0)),
        out_shape=jax.ShapeDtypeStruct((C, NP, F), jnp.float32),
    )(h, a3, c3, disb)


def _head_body(h_ref, gamma_ref, beta_ref, batch_ref, wp_ref, bp_ref,
               out_ref):
    c = pl.program_id(0)
    d = h_ref[0]
    s = jnp.sum(d, axis=0)
    q = jnp.sum(d * d, axis=0)
    mu = s / float(N)
    var = q / float(N) - mu * mu
    a2 = gamma_ref[0, 0] * lax.rsqrt(var + 1e-5)
    c2 = beta_ref[0, 0] - mu * a2
    iota_g = lax.broadcasted_iota(jnp.int32, (NUM_GRAPHS, NP), 0)
    oh = (batch_ref[0, :][None, :] == iota_g).astype(jnp.float32)
    psum = lax.dot_general(oh, d, (((1,), (0,)), ((), ())),
                           preferred_element_type=jnp.float32)
    cnt = jnp.sum(oh, axis=1)[:, None]
    pb = psum * a2[None, :] + cnt * c2[None, :]
    pr = lax.dot_general(pb, wp_ref[0], (((1,), (1,)), ((), ())),
                         preferred_element_type=jnp.float32)

    @pl.when(c == 0)
    def _():
        out_ref[...] = jnp.broadcast_to(bp_ref[0, 0][None, :],
                                        (NUM_GRAPHS, F))
    out_ref[...] += pr


def _head(h2, gamma3, beta3, batch2d, wp_chunks, bp3):
    return pl.pallas_call(
        _head_body,
        grid=(C,),
        in_specs=[
            pl.BlockSpec((1, NP, F), lambda c: (c, 0, 0)),
            pl.BlockSpec((1, 1, F), lambda c: (c, 0, 0)),
            pl.BlockSpec((1, 1, F), lambda c: (c, 0, 0)),
            pl.BlockSpec((8, NP), lambda c: (0, 0)),
            pl.BlockSpec((1, F, F), lambda c: (c, 0, 0)),
            pl.BlockSpec((1, 1, F), lambda c: (0, 0, 0)),
        ],
        out_specs=pl.BlockSpec((NUM_GRAPHS, F), lambda c: (0, 0)),
        out_shape=jax.ShapeDtypeStruct((NUM_GRAPHS, F), jnp.float32),
    )(h2, gamma3, beta3, batch2d, wp_chunks, bp3)


def _topk_body(w_ref, out_ref):
    w = w_ref[...]
    iota_lin = (lax.broadcasted_iota(jnp.int32, (C, F), 0) * F +
                lax.broadcasted_iota(jnp.int32, (C, F), 1))
    oidx = lax.broadcasted_iota(jnp.int32, (8, 128), 0) * 128 + \
        lax.broadcasted_iota(jnp.int32, (8, 128), 1)
    out = jnp.zeros((8, 128), jnp.int32)
    for k in range(K):
        m = jnp.max(w)
        idx = jnp.min(jnp.where(w == m, iota_lin, jnp.int32(2**30)))
        out = jnp.where(oidx == k, idx, out)
        w = jnp.where(iota_lin == idx, -jnp.inf, w)
    out_ref[...] = out


def _topk(w_pad2):
    return pl.pallas_call(
        _topk_body,
        out_shape=jax.ShapeDtypeStruct((8, 128), jnp.int32),
    )(w_pad2)


def kernel(x, edge_index, batch, W, W_emb, b_emb, root0, root1,
           gamma0, beta0, gamma1, beta1, Wp, bp):
    f32 = jnp.float32
    x_pad = jnp.pad(x, ((0, NP - N), (0, 0)))
    wemb_pad = jnp.pad(W_emb, ((0, EMB_P - EMB), (0, 0)))
    scale3 = jnp.pad(W, (0, EMB_P - EMB)).reshape(C, 1, F)
    bias3 = jnp.pad(b_emb, (0, EMB_P - EMB)).reshape(C, 1, F)
    root0c = jnp.pad(root0, (0, EMB_P - EMB)).reshape(C, 1, F)
    root1c = jnp.pad(root1, (0, EMB_P - EMB)).reshape(C, 1, F)
    g0 = jnp.pad(gamma0, (0, EMB_P - EMB)).reshape(C, 1, F)
    b0 = jnp.pad(beta0, (0, EMB_P - EMB)).reshape(C, 1, F)
    g1 = jnp.pad(gamma1, (0, EMB_P - EMB)).reshape(C, 1, F)
    b1 = jnp.pad(beta1, (0, EMB_P - EMB)).reshape(C, 1, F)

    npad = EP - E
    row_p = jnp.concatenate([edge_index[0], jnp.zeros((npad,), jnp.int32)])
    col_p = jnp.concatenate(
        [edge_index[1], N + (jnp.arange(npad, dtype=jnp.int32) % (NP - N))])
    row3d = row_p.reshape(NT, NBM, EB)
    col3d = col_p.reshape(NT, NBM, EB)

    batch_pad = jnp.concatenate(
        [batch, jnp.full((NP - N,), NUM_GRAPHS, jnp.int32)])
    batch2d = jnp.broadcast_to(batch_pad[None, :], (8, NP))

    wp_pad = jnp.pad(Wp, ((0, F - OUT), (0, EMB_P - EMB)))
    wp_chunks = jnp.transpose(wp_pad.reshape(F, C, F), (1, 0, 2))
    bp3 = jnp.pad(bp, (0, F - OUT)).reshape(1, 1, F)

    w_pad2 = jnp.pad(W, (0, EMB_P - EMB),
                     constant_values=-jnp.inf).reshape(C, F)

    deg = _k1(col3d)
    inv2d, s2d, dis2d = _degpost(deg.reshape(NP // 128, 128))
    inv1 = inv2d.reshape(NP)
    s1 = s2d.reshape(NP)
    dis1 = dis2d.reshape(NP)
    disb = jnp.broadcast_to(dis1[:, None], (NP, F))

    h0p, t0p = _emb_matmul(x_pad, wemb_pad, scale3, bias3, disb)
    out1 = _conv(t0p, h0p, row3d, col3d, inv1, s1, dis1, root0c)
    a1, c1 = _stats_affine(out1, g0, b0)
    t1p = _t1(out1, a1, c1, disb)
    out2 = _conv(t1p, t1p, row3d, col3d, inv1, s1, dis1, root1c)
    pred_pad = _head(out2, g1, b1, batch2d, wp_chunks, bp3)
    topk_pad = _topk(w_pad2)

    pred = pred_pad[:, :OUT]
    topk_indices = topk_pad.reshape(-1)[:K]
    return pred, topk_indices
